# baseline (speedup 1.0000x reference)
import functools

import jax
import jax.numpy as jnp
from jax import lax
from jax.experimental import pallas as pl
from jax.experimental.pallas import tpu as pltpu
from jax.experimental.pallas import tpu_sc as plsc

N = 10000
E = 320000
F = 128
OUT_F = 16
EPS = 1e-5

NC = 2
NS = 16
NW = NC * NS
K = 128
GRP = 8
C = 80
G = C // GRP
E_PAD = NW * C * K
N2 = 12288
N3 = 10240
RPS = N3 // NS
DR = N2 // NS
BN = 2000

_mesh = plsc.VectorSubcoreMesh(core_axis_name="c", subcore_axis_name="s",
                               num_cores=NC, num_subcores=NS)


def _wid():
    return lax.axis_index("s") * NC + lax.axis_index("c")


def _fill_zero(ref, rows, width):
    zv = jnp.zeros((16,), jnp.float32)

    def body(i, carry):
        for j in range(width // 16):
            ref[i, pl.ds(j * 16, 16)] = zv
        return carry
    lax.fori_loop(0, rows, body, 0)


@functools.partial(
    pl.kernel,
    out_type=jax.ShapeDtypeStruct((N3,), jnp.float32),
    mesh=_mesh,
    compiler_params=pltpu.CompilerParams(needs_layout_passes=False),
    scratch_types=[
        pltpu.VMEM((640,), jnp.float32),
        pltpu.VMEM((640,), jnp.float32),
        pltpu.VMEM((640,), jnp.float32),
    ],
)
def _dinv_kernel(d0_hbm, d1_hbm, out_hbm, v0, v1, vo):
    wid = _wid()

    @pl.when(wid < 16)
    def _():
        base = wid * 640
        pltpu.sync_copy(d0_hbm.at[pl.ds(base, 640)], v0)
        pltpu.sync_copy(d1_hbm.at[pl.ds(base, 640)], v1)

        def grp(g, carry):
            sl = pl.ds(g * 16, 16)
            d = v0[sl] + v1[sl]
            i = lax.bitcast_convert_type(d, jnp.int32)
            i = jnp.int32(0x5F3759DF) - lax.shift_right_arithmetic(i, 1)
            y = lax.bitcast_convert_type(i, jnp.float32)
            for _ in range(4):
                y = y * (1.5 - 0.5 * d * y * y)
            vo[sl] = jnp.where(d > 0.0, y, 0.0)
            return carry
        lax.fori_loop(0, 40, grp, 0)
        pltpu.sync_copy(vo, out_hbm.at[pl.ds(base, 640)])


@functools.partial(
    pl.kernel,
    out_type=jax.ShapeDtypeStruct((NW, C, K), jnp.float32),
    mesh=_mesh,
    compiler_params=pltpu.CompilerParams(needs_layout_passes=False),
    scratch_types=[
        pltpu.VMEM((C, K), jnp.int32),
        pltpu.VMEM((C, K), jnp.int32),
        pltpu.VMEM((C, K), jnp.float32),
        pltpu.VMEM((C, K), jnp.float32),
        pltpu.VMEM((N3,), jnp.float32),
    ],
)
def _val_kernel(srcs_hbm, dsts_hbm, ew_hbm, dinv_hbm, out_hbm,
                srcs_v, dsts_v, ew_v, vout_v, dv_v):
    wid = _wid()
    pltpu.sync_copy(srcs_hbm.at[wid], srcs_v)
    pltpu.sync_copy(dsts_hbm.at[wid], dsts_v)
    pltpu.sync_copy(ew_hbm.at[wid], ew_v)
    pltpu.sync_copy(dinv_hbm, dv_v)

    def chunk(c, carry):
        def grp(g, carry2):
            sl = pl.ds(g * 16, 16)
            sv = srcs_v[c, sl]
            dv = dsts_v[c, sl]
            dsq = plsc.load_gather(dv_v, [sv])
            ddq = plsc.load_gather(dv_v, [dv])
            vout_v[c, sl] = -(dsq * ew_v[c, sl] * ddq)
            return carry2
        lax.fori_loop(0, K // 16, grp, 0)
        return carry
    lax.fori_loop(0, C, chunk, 0)
    pltpu.sync_copy(vout_v, out_hbm.at[wid])


@functools.partial(
    pl.kernel,
    out_type=jax.ShapeDtypeStruct((NC, N3, F), jnp.float32),
    mesh=_mesh,
    compiler_params=pltpu.CompilerParams(needs_layout_passes=False),
    scratch_types=[
        pltpu.VMEM((C, K), jnp.int32),
        pltpu.VMEM((GRP, K), jnp.int32),
        pltpu.VMEM((GRP, K), jnp.int32),
        pltpu.VMEM((GRP, K), jnp.float32),
        pltpu.VMEM((GRP, K), jnp.float32),
        pltpu.VMEM((K, F), jnp.float32),
        pltpu.VMEM((K, F), jnp.float32),
        pltpu.SemaphoreType.DMA,
        pltpu.SemaphoreType.DMA,
        pltpu.SemaphoreType.DMA,
        pltpu.SemaphoreType.DMA,
        pltpu.SemaphoreType.DMA,
        pltpu.SemaphoreType.DMA,
        pltpu.VMEM_SHARED((N3, F), jnp.float32),
    ],
)
def _prop_kernel(h_hbm, srcs_hbm, dsts_hbm, valr_hbm, out_hbm,
                 srcs_v, dstg0, dstg1, valg0, valg1, rows0, rows1,
                 semg0, semg1, sems0, sems1, semgrp0, semgrp1, acc):
    cid = lax.axis_index("c")
    sid = lax.axis_index("s")
    wid = _wid()
    rows = (rows0, rows1)
    semg = (semg0, semg1)
    sems = (sems0, sems1)
    dstg = (dstg0, dstg1)
    valg = (valg0, valg1)

    _fill_zero(rows0, K, F)
    for kk in range(RPS // K):
        pltpu.sync_copy(rows0, acc.at[pl.ds(sid * RPS + kk * K, K)])
    plsc.subcore_barrier()

    pltpu.sync_copy(srcs_hbm.at[wid], srcs_v)
    pltpu.async_copy(dsts_hbm.at[wid, pl.ds(0, GRP)], dstg0, semgrp0)
    pltpu.async_copy(valr_hbm.at[wid, pl.ds(0, GRP)], valg0, semgrp0)
    pltpu.async_copy(dsts_hbm.at[wid, pl.ds(GRP, GRP)], dstg1, semgrp1)
    pltpu.async_copy(valr_hbm.at[wid, pl.ds(GRP, GRP)], valg1, semgrp1)
    pltpu.async_copy(h_hbm.at[srcs_v.at[0]], rows0, semg0)

    def wait_group(gb):
        pltpu.make_async_copy(dsts_hbm.at[wid, pl.ds(0, GRP)], dstg[gb],
                              semgrp[gb]).wait()
        pltpu.make_async_copy(valr_hbm.at[wid, pl.ds(0, GRP)], valg[gb],
                              semgrp[gb]).wait()

    semgrp = (semgrp0, semgrp1)

    def do_group(t, gp, gb):
        base = gp * GRP

        def wait_gather(b, c):
            pltpu.make_async_copy(h_hbm.at[srcs_v.at[c]], rows[b],
                                  semg[b]).wait()

        wait_group(gb)
        for i in range(GRP):
            b = i % 2
            c = base + i
            wait_gather(b, c)
            nxt = lax.rem(c + 1, C)
            pltpu.async_copy(h_hbm.at[srcs_v.at[nxt]], rows[1 - b],
                             semg[1 - b])

            @pl.when(c >= 2)
            def _():
                pltpu.make_async_copy(rows[b], acc.at[dstg[gb].at[i]],
                                      sems[b]).wait()

            def row(r4, carry2):
                for u in range(4):
                    r = r4 * 4 + u
                    vv = plsc.load_gather(valg[gb].at[i],
                                          [jnp.full((16,), 1, jnp.int32) * r])
                    for j in range(F // 16):
                        sl = pl.ds(j * 16, 16)
                        rows[b][r, sl] = rows[b][r, sl] * vv
                return carry2
            lax.fori_loop(0, K // 4, row, 0)
            pltpu.async_copy(rows[b], acc.at[dstg[gb].at[i]], sems[b],
                             add=True)
        nxtg = lax.rem(gp + 2, G)
        pltpu.async_copy(dsts_hbm.at[wid, pl.ds(nxtg * GRP, GRP)], dstg[gb],
                         semgrp[gb])
        pltpu.async_copy(valr_hbm.at[wid, pl.ds(nxtg * GRP, GRP)], valg[gb],
                         semgrp[gb])

    def pair(t, carry):
        do_group(t, 2 * t, 0)
        do_group(t, 2 * t + 1, 1)
        return carry
    lax.fori_loop(0, G // 2, pair, 0)

    pltpu.make_async_copy(h_hbm.at[srcs_v.at[0]], rows0, semg0).wait()
    pltpu.make_async_copy(rows0, acc.at[dstg0.at[0]], sems0).wait()
    pltpu.make_async_copy(rows1, acc.at[dstg1.at[0]], sems1).wait()
    wait_group(0)
    wait_group(1)

    plsc.subcore_barrier()
    for kk in range(RPS // 128):
        r0 = sid * RPS + kk * 128
        pltpu.sync_copy(acc.at[pl.ds(r0, 128)], out_hbm.at[cid, pl.ds(r0, 128)])


def _sum2_body(p_ref, o_ref):
    o_ref[...] = p_ref[0] + p_ref[1]


def _sum2(p):
    return pl.pallas_call(
        _sum2_body,
        grid=(N // BN,),
        in_specs=[pl.BlockSpec((NC, BN, F), lambda i: (0, i, 0))],
        out_specs=pl.BlockSpec((BN, F), lambda i: (i, 0)),
        out_shape=jax.ShapeDtypeStruct((N, F), jnp.float32),
    )(p)


def _layer_body(t0_ref, t1_ref, q_ref, w0_ref, w1_ref, w2_ref, b_ref,
                s_ref, t_ref, o_ref):
    t0 = t0_ref[...]
    t2 = 2.0 * (q_ref[0] + q_ref[1]) - t0
    acc = jnp.dot(t0, w0_ref[...], preferred_element_type=jnp.float32)
    acc += jnp.dot(t1_ref[...], w1_ref[...], preferred_element_type=jnp.float32)
    acc += jnp.dot(t2, w2_ref[...], preferred_element_type=jnp.float32)
    acc += b_ref[...]
    h = jnp.maximum(acc, 0.0)
    o_ref[...] = h * s_ref[...] + t_ref[...]


def _final_body(t0_ref, t1_ref, q_ref, w0_ref, w1_ref, w2_ref, b_ref,
                s_ref, t_ref, lw_ref, lb_ref, o_ref):
    t0 = t0_ref[...]
    t2 = 2.0 * (q_ref[0] + q_ref[1]) - t0
    acc = jnp.dot(t0, w0_ref[...], preferred_element_type=jnp.float32)
    acc += jnp.dot(t1_ref[...], w1_ref[...], preferred_element_type=jnp.float32)
    acc += jnp.dot(t2, w2_ref[...], preferred_element_type=jnp.float32)
    acc += b_ref[...]
    h = jnp.maximum(acc, 0.0)
    h = h * s_ref[...] + t_ref[...]
    o_ref[...] = jnp.dot(h, lw_ref[...], preferred_element_type=jnp.float32) + lb_ref[...]


def _row_spec():
    return pl.BlockSpec((BN, F), lambda i: (i, 0))


def _full_spec(shape):
    return pl.BlockSpec(shape, lambda i: tuple(0 for _ in shape))


def _cheb_layer(t0, t1, q, W, b, bn_w, bn_b, lin_w=None, lin_b=None):
    s = (bn_w / jnp.sqrt(1.0 + EPS)).reshape(1, F)
    t = bn_b.reshape(1, F)
    b2d = b.reshape(1, F)
    grid = (N // BN,)
    common = [_row_spec(), _row_spec(),
              pl.BlockSpec((NC, BN, F), lambda i: (0, i, 0)),
              _full_spec((F, F)), _full_spec((F, F)), _full_spec((F, F)),
              _full_spec((1, F)), _full_spec((1, F)), _full_spec((1, F))]
    if lin_w is None:
        return pl.pallas_call(
            _layer_body,
            grid=grid,
            in_specs=common,
            out_specs=_row_spec(),
            out_shape=jax.ShapeDtypeStruct((N, F), jnp.float32),
        )(t0, t1, q, W[0], W[1], W[2], b2d, s, t)
    lwT = lin_w.T
    lb2d = lin_b.reshape(1, OUT_F)
    return pl.pallas_call(
        _final_body,
        grid=grid,
        in_specs=common + [_full_spec((F, OUT_F)), _full_spec((1, OUT_F))],
        out_specs=pl.BlockSpec((BN, OUT_F), lambda i: (i, 0)),
        out_shape=jax.ShapeDtypeStruct((N, OUT_F), jnp.float32),
    )(t0, t1, q, W[0], W[1], W[2], b2d, s, t, lwT, lb2d)


def kernel(x, edge_index, edge_weight, W1, b1, bn1_w, bn1_b, W2, b2,
           bn2_w, bn2_b, lin_w, lin_b):
    pad = E_PAD - E
    srcs = jnp.pad(edge_index[0], (0, pad)).reshape(NW, C, K)
    dsts = jnp.pad(edge_index[1], (0, pad)).reshape(NW, C, K)
    ewf = jnp.pad(edge_weight, (0, pad)).reshape(NW, C, K)
    ewr = jnp.asarray(jnp.broadcast_to(ewf[..., None], (NW, C, K, 16)))

    ones = jnp.ones((N, F), jnp.float32)
    p_deg = _prop_kernel(ones, srcs, srcs, ewf)
    dinvf = _dinv_kernel(p_deg[0, :, 0], p_deg[1, :, 0])
    valr = _val_kernel(srcs, dsts, ewf, dinvf)

    def prop(h):
        return _prop_kernel(h, srcs, dsts, valr)

    p = prop(x)
    t1 = _sum2(p)
    q = prop(t1)
    h = _cheb_layer(x, t1, q, W1, b1, bn1_w, bn1_b)
    p = prop(h)
    t1 = _sum2(p)
    q = prop(t1)
    return _cheb_layer(h, t1, q, W2, b2, bn2_w, bn2_b, lin_w, lin_b)


# speedup vs baseline: 1.2046x; 1.2046x over previous
import functools

import jax
import jax.numpy as jnp
from jax import lax
from jax.experimental import pallas as pl
from jax.experimental.pallas import tpu as pltpu
from jax.experimental.pallas import tpu_sc as plsc

N = 10000
E = 320000
F = 128
OUT_F = 16
EPS = 1e-5

NC = 2
NS = 16
NW = NC * NS
K = 128
GRP = 8
C = 80
G = C // GRP
E_PAD = NW * C * K
N2 = 12288
N3 = 10240
RPS = N3 // NS
DR = N2 // NS
BN = 2000

_mesh = plsc.VectorSubcoreMesh(core_axis_name="c", subcore_axis_name="s",
                               num_cores=NC, num_subcores=NS)


def _wid():
    return lax.axis_index("s") * NC + lax.axis_index("c")


def _fill_zero(ref, rows, width):
    zv = jnp.zeros((16,), jnp.float32)

    def body(i, carry):
        for j in range(width // 16):
            ref[i, pl.ds(j * 16, 16)] = zv
        return carry
    lax.fori_loop(0, rows, body, 0)


@functools.partial(
    pl.kernel,
    out_type=jax.ShapeDtypeStruct((N3,), jnp.float32),
    mesh=_mesh,
    compiler_params=pltpu.CompilerParams(needs_layout_passes=False),
    scratch_types=[
        pltpu.VMEM((640,), jnp.float32),
        pltpu.VMEM((640,), jnp.float32),
        pltpu.VMEM((640,), jnp.float32),
    ],
)
def _dinv_kernel(d0_hbm, d1_hbm, out_hbm, v0, v1, vo):
    wid = _wid()

    @pl.when(wid < 16)
    def _():
        base = wid * 640
        pltpu.sync_copy(d0_hbm.at[pl.ds(base, 640)], v0)
        pltpu.sync_copy(d1_hbm.at[pl.ds(base, 640)], v1)

        def grp(g, carry):
            sl = pl.ds(g * 16, 16)
            d = v0[sl] + v1[sl]
            i = lax.bitcast_convert_type(d, jnp.int32)
            i = jnp.int32(0x5F3759DF) - lax.shift_right_arithmetic(i, 1)
            y = lax.bitcast_convert_type(i, jnp.float32)
            for _ in range(4):
                y = y * (1.5 - 0.5 * d * y * y)
            vo[sl] = jnp.where(d > 0.0, y, 0.0)
            return carry
        lax.fori_loop(0, 40, grp, 0)
        pltpu.sync_copy(vo, out_hbm.at[pl.ds(base, 640)])


@functools.partial(
    pl.kernel,
    out_type=jax.ShapeDtypeStruct((NW, C, K), jnp.float32),
    mesh=_mesh,
    compiler_params=pltpu.CompilerParams(needs_layout_passes=False),
    scratch_types=[
        pltpu.VMEM((C, K), jnp.int32),
        pltpu.VMEM((C, K), jnp.int32),
        pltpu.VMEM((C, K), jnp.float32),
        pltpu.VMEM((C, K), jnp.float32),
        pltpu.VMEM((N3,), jnp.float32),
    ],
)
def _val_kernel(srcs_hbm, dsts_hbm, ew_hbm, dinv_hbm, out_hbm,
                srcs_v, dsts_v, ew_v, vout_v, dv_v):
    wid = _wid()
    pltpu.sync_copy(srcs_hbm.at[wid], srcs_v)
    pltpu.sync_copy(dsts_hbm.at[wid], dsts_v)
    pltpu.sync_copy(ew_hbm.at[wid], ew_v)
    pltpu.sync_copy(dinv_hbm, dv_v)

    def chunk(c, carry):
        def grp(g, carry2):
            sl = pl.ds(g * 16, 16)
            sv = srcs_v[c, sl]
            dv = dsts_v[c, sl]
            dsq = plsc.load_gather(dv_v, [sv])
            ddq = plsc.load_gather(dv_v, [dv])
            vout_v[c, sl] = -(dsq * ew_v[c, sl] * ddq)
            return carry2
        lax.fori_loop(0, K // 16, grp, 0)
        return carry
    lax.fori_loop(0, C, chunk, 0)
    pltpu.sync_copy(vout_v, out_hbm.at[wid])


@functools.partial(
    pl.kernel,
    out_type=jax.ShapeDtypeStruct((NC, N3, F), jnp.float32),
    mesh=_mesh,
    compiler_params=pltpu.CompilerParams(needs_layout_passes=False),
    scratch_types=[
        pltpu.VMEM((C, K), jnp.int32),
        pltpu.VMEM((GRP, K), jnp.int32),
        pltpu.VMEM((GRP, K), jnp.int32),
        pltpu.VMEM((GRP, K), jnp.float32),
        pltpu.VMEM((GRP, K), jnp.float32),
        pltpu.VMEM((K, F), jnp.float32),
        pltpu.VMEM((K, F), jnp.float32),
        pltpu.SemaphoreType.DMA,
        pltpu.SemaphoreType.DMA,
        pltpu.SemaphoreType.DMA,
        pltpu.SemaphoreType.DMA,
        pltpu.SemaphoreType.DMA,
        pltpu.SemaphoreType.DMA,
        pltpu.VMEM_SHARED((N3, F), jnp.float32),
    ],
)
def _prop_kernel(h_hbm, srcs_hbm, dsts_hbm, valr_hbm, out_hbm,
                 srcs_v, dstg0, dstg1, valg0, valg1, rows0, rows1,
                 semg0, semg1, sems0, sems1, semgrp0, semgrp1, acc):
    cid = lax.axis_index("c")
    sid = lax.axis_index("s")
    wid = _wid()
    rows = (rows0, rows1)
    semg = (semg0, semg1)
    sems = (sems0, sems1)
    dstg = (dstg0, dstg1)
    valg = (valg0, valg1)

    _fill_zero(rows0, K, F)
    for kk in range(RPS // K):
        pltpu.sync_copy(rows0, acc.at[pl.ds(sid * RPS + kk * K, K)])
    plsc.subcore_barrier()

    pltpu.sync_copy(srcs_hbm.at[wid], srcs_v)
    pltpu.async_copy(dsts_hbm.at[wid, pl.ds(0, GRP)], dstg0, semgrp0)
    pltpu.async_copy(valr_hbm.at[wid, pl.ds(0, GRP)], valg0, semgrp0)
    pltpu.async_copy(dsts_hbm.at[wid, pl.ds(GRP, GRP)], dstg1, semgrp1)
    pltpu.async_copy(valr_hbm.at[wid, pl.ds(GRP, GRP)], valg1, semgrp1)
    pltpu.async_copy(h_hbm.at[srcs_v.at[0]], rows0, semg0)

    def wait_group(gb):
        pltpu.make_async_copy(dsts_hbm.at[wid, pl.ds(0, GRP)], dstg[gb],
                              semgrp[gb]).wait()
        pltpu.make_async_copy(valr_hbm.at[wid, pl.ds(0, GRP)], valg[gb],
                              semgrp[gb]).wait()

    semgrp = (semgrp0, semgrp1)

    def do_group(t, gp, gb):
        base = gp * GRP

        def wait_gather(b, c):
            pltpu.make_async_copy(h_hbm.at[srcs_v.at[c]], rows[b],
                                  semg[b]).wait()

        wait_group(gb)
        for i in range(GRP):
            b = i % 2
            c = base + i
            wait_gather(b, c)
            nxt = lax.rem(c + 1, C)
            pltpu.async_copy(h_hbm.at[srcs_v.at[nxt]], rows[1 - b],
                             semg[1 - b])

            @pl.when(c >= 2)
            def _():
                pltpu.make_async_copy(rows[b], acc.at[dstg[gb].at[i]],
                                      sems[b]).wait()

            def row(r, carry2):
                vv = plsc.load_gather(valg[gb].at[i],
                                      [jnp.full((16,), 1, jnp.int32) * r])
                for j in range(F // 16):
                    sl = pl.ds(j * 16, 16)
                    rows[b][r, sl] = rows[b][r, sl] * vv
                return carry2
            lax.fori_loop(0, K, row, 0)
            pltpu.async_copy(rows[b], acc.at[dstg[gb].at[i]], sems[b],
                             add=True)
        nxtg = lax.rem(gp + 2, G)
        pltpu.async_copy(dsts_hbm.at[wid, pl.ds(nxtg * GRP, GRP)], dstg[gb],
                         semgrp[gb])
        pltpu.async_copy(valr_hbm.at[wid, pl.ds(nxtg * GRP, GRP)], valg[gb],
                         semgrp[gb])

    def pair(t, carry):
        do_group(t, 2 * t, 0)
        do_group(t, 2 * t + 1, 1)
        return carry
    lax.fori_loop(0, G // 2, pair, 0)

    pltpu.make_async_copy(h_hbm.at[srcs_v.at[0]], rows0, semg0).wait()
    pltpu.make_async_copy(rows0, acc.at[dstg0.at[0]], sems0).wait()
    pltpu.make_async_copy(rows1, acc.at[dstg1.at[0]], sems1).wait()
    wait_group(0)
    wait_group(1)

    plsc.subcore_barrier()
    for kk in range(RPS // 128):
        r0 = sid * RPS + kk * 128
        pltpu.sync_copy(acc.at[pl.ds(r0, 128)], out_hbm.at[cid, pl.ds(r0, 128)])


@functools.partial(
    pl.kernel,
    out_type=jax.ShapeDtypeStruct((NC, N3, F), jnp.float32),
    mesh=_mesh,
    compiler_params=pltpu.CompilerParams(needs_layout_passes=False),
    scratch_types=[
        pltpu.VMEM((GRP, K), jnp.int32),
        pltpu.VMEM((GRP, K), jnp.int32),
        pltpu.VMEM((GRP, K), jnp.float32),
        pltpu.VMEM((GRP, K), jnp.float32),
        pltpu.VMEM((K, F), jnp.float32),
        pltpu.VMEM((K, F), jnp.float32),
        pltpu.SemaphoreType.DMA,
        pltpu.SemaphoreType.DMA,
        pltpu.SemaphoreType.DMA,
        pltpu.SemaphoreType.DMA,
        pltpu.VMEM_SHARED((N3, F), jnp.float32),
    ],
)
def _degp_kernel(srcs_hbm, valr_hbm, out_hbm,
                 dstg0, dstg1, valg0, valg1, rows0, rows1,
                 sems0, sems1, semgrp0, semgrp1, acc):
    cid = lax.axis_index("c")
    sid = lax.axis_index("s")
    wid = _wid()
    rows = (rows0, rows1)
    sems = (sems0, sems1)
    dstg = (dstg0, dstg1)
    valg = (valg0, valg1)
    semgrp = (semgrp0, semgrp1)

    _fill_zero(rows0, K, F)
    for kk in range(RPS // K):
        pltpu.sync_copy(rows0, acc.at[pl.ds(sid * RPS + kk * K, K)])
    plsc.subcore_barrier()

    pltpu.async_copy(srcs_hbm.at[wid, pl.ds(0, GRP)], dstg0, semgrp0)
    pltpu.async_copy(valr_hbm.at[wid, pl.ds(0, GRP)], valg0, semgrp0)
    pltpu.async_copy(srcs_hbm.at[wid, pl.ds(GRP, GRP)], dstg1, semgrp1)
    pltpu.async_copy(valr_hbm.at[wid, pl.ds(GRP, GRP)], valg1, semgrp1)

    def wait_group(gb):
        pltpu.make_async_copy(srcs_hbm.at[wid, pl.ds(0, GRP)], dstg[gb],
                              semgrp[gb]).wait()
        pltpu.make_async_copy(valr_hbm.at[wid, pl.ds(0, GRP)], valg[gb],
                              semgrp[gb]).wait()

    def do_group(gp, gb):
        wait_group(gb)
        for i in range(GRP):
            b = i % 2
            c = gp * GRP + i

            @pl.when(c >= 2)
            def _():
                pltpu.make_async_copy(rows[b], acc.at[dstg[gb].at[i]],
                                      sems[b]).wait()

            def row(r, carry2):
                vv = plsc.load_gather(valg[gb].at[i],
                                      [jnp.full((16,), 1, jnp.int32) * r])
                for j in range(F // 16):
                    rows[b][r, pl.ds(j * 16, 16)] = vv
                return carry2
            lax.fori_loop(0, K, row, 0)
            pltpu.async_copy(rows[b], acc.at[dstg[gb].at[i]], sems[b],
                             add=True)
        nxtg = lax.rem(gp + 2, G)
        pltpu.async_copy(srcs_hbm.at[wid, pl.ds(nxtg * GRP, GRP)], dstg[gb],
                         semgrp[gb])
        pltpu.async_copy(valr_hbm.at[wid, pl.ds(nxtg * GRP, GRP)], valg[gb],
                         semgrp[gb])

    def pair(t, carry):
        do_group(2 * t, 0)
        do_group(2 * t + 1, 1)
        return carry
    lax.fori_loop(0, G // 2, pair, 0)

    pltpu.make_async_copy(rows0, acc.at[dstg0.at[0]], sems0).wait()
    pltpu.make_async_copy(rows1, acc.at[dstg1.at[0]], sems1).wait()
    wait_group(0)
    wait_group(1)

    plsc.subcore_barrier()
    for kk in range(RPS // 128):
        r0 = sid * RPS + kk * 128
        pltpu.sync_copy(acc.at[pl.ds(r0, 128)], out_hbm.at[cid, pl.ds(r0, 128)])


def _sum2_body(p_ref, o_ref):
    o_ref[...] = p_ref[0] + p_ref[1]


def _sum2(p):
    return pl.pallas_call(
        _sum2_body,
        grid=(N // BN,),
        in_specs=[pl.BlockSpec((NC, BN, F), lambda i: (0, i, 0))],
        out_specs=pl.BlockSpec((BN, F), lambda i: (i, 0)),
        out_shape=jax.ShapeDtypeStruct((N, F), jnp.float32),
    )(p)


def _layer_body(t0_ref, t1_ref, q_ref, w0_ref, w1_ref, w2_ref, b_ref,
                s_ref, t_ref, o_ref):
    t0 = t0_ref[...]
    t2 = 2.0 * (q_ref[0] + q_ref[1]) - t0
    acc = jnp.dot(t0, w0_ref[...], preferred_element_type=jnp.float32)
    acc += jnp.dot(t1_ref[...], w1_ref[...], preferred_element_type=jnp.float32)
    acc += jnp.dot(t2, w2_ref[...], preferred_element_type=jnp.float32)
    acc += b_ref[...]
    h = jnp.maximum(acc, 0.0)
    o_ref[...] = h * s_ref[...] + t_ref[...]


def _final_body(t0_ref, t1_ref, q_ref, w0_ref, w1_ref, w2_ref, b_ref,
                s_ref, t_ref, lw_ref, lb_ref, o_ref):
    t0 = t0_ref[...]
    t2 = 2.0 * (q_ref[0] + q_ref[1]) - t0
    acc = jnp.dot(t0, w0_ref[...], preferred_element_type=jnp.float32)
    acc += jnp.dot(t1_ref[...], w1_ref[...], preferred_element_type=jnp.float32)
    acc += jnp.dot(t2, w2_ref[...], preferred_element_type=jnp.float32)
    acc += b_ref[...]
    h = jnp.maximum(acc, 0.0)
    h = h * s_ref[...] + t_ref[...]
    o_ref[...] = jnp.dot(h, lw_ref[...], preferred_element_type=jnp.float32) + lb_ref[...]


def _row_spec():
    return pl.BlockSpec((BN, F), lambda i: (i, 0))


def _full_spec(shape):
    return pl.BlockSpec(shape, lambda i: tuple(0 for _ in shape))


def _cheb_layer(t0, t1, q, W, b, bn_w, bn_b, lin_w=None, lin_b=None):
    s = (bn_w / jnp.sqrt(1.0 + EPS)).reshape(1, F)
    t = bn_b.reshape(1, F)
    b2d = b.reshape(1, F)
    grid = (N // BN,)
    common = [_row_spec(), _row_spec(),
              pl.BlockSpec((NC, BN, F), lambda i: (0, i, 0)),
              _full_spec((F, F)), _full_spec((F, F)), _full_spec((F, F)),
              _full_spec((1, F)), _full_spec((1, F)), _full_spec((1, F))]
    if lin_w is None:
        return pl.pallas_call(
            _layer_body,
            grid=grid,
            in_specs=common,
            out_specs=_row_spec(),
            out_shape=jax.ShapeDtypeStruct((N, F), jnp.float32),
        )(t0, t1, q, W[0], W[1], W[2], b2d, s, t)
    lwT = lin_w.T
    lb2d = lin_b.reshape(1, OUT_F)
    return pl.pallas_call(
        _final_body,
        grid=grid,
        in_specs=common + [_full_spec((F, OUT_F)), _full_spec((1, OUT_F))],
        out_specs=pl.BlockSpec((BN, OUT_F), lambda i: (i, 0)),
        out_shape=jax.ShapeDtypeStruct((N, OUT_F), jnp.float32),
    )(t0, t1, q, W[0], W[1], W[2], b2d, s, t, lwT, lb2d)


def kernel(x, edge_index, edge_weight, W1, b1, bn1_w, bn1_b, W2, b2,
           bn2_w, bn2_b, lin_w, lin_b):
    pad = E_PAD - E
    srcs = jnp.pad(edge_index[0], (0, pad)).reshape(NW, C, K)
    dsts = jnp.pad(edge_index[1], (0, pad)).reshape(NW, C, K)
    ewf = jnp.pad(edge_weight, (0, pad)).reshape(NW, C, K)
    ewr = jnp.asarray(jnp.broadcast_to(ewf[..., None], (NW, C, K, 16)))

    p_deg = _degp_kernel(srcs, ewf)
    dinvf = _dinv_kernel(p_deg[0, :, 0], p_deg[1, :, 0])
    valr = _val_kernel(srcs, dsts, ewf, dinvf)

    def prop(h):
        return _prop_kernel(h, srcs, dsts, valr)

    p = prop(x)
    t1 = _sum2(p)
    q = prop(t1)
    h = _cheb_layer(x, t1, q, W1, b1, bn1_w, bn1_b)
    p = prop(h)
    t1 = _sum2(p)
    q = prop(t1)
    return _cheb_layer(h, t1, q, W2, b2, bn2_w, bn2_b, lin_w, lin_b)
